# Initial kernel scaffold; baseline (speedup 1.0000x reference)
import jax
import jax.numpy as jnp
from jax.experimental import pallas as pl


def kernel(x, pos, params):
    raise NotImplementedError("write your pallas kernel here")


# speedup vs baseline: 164.9290x; 164.9290x over previous
import functools

import jax
import jax.numpy as jnp
import numpy as np
from jax import lax
from jax.experimental import pallas as pl
from jax.experimental.pallas import tpu as pltpu
from jax.experimental.pallas import tpu_sc as plsc

KS = 16
NB_BATCH = 4


def _pad128(c):
    return ((c + 127) // 128) * 128


def _knn_body(qT_ref, pL_ref, o_ref, *, n):
    q = qT_ref[0]
    p = pL_ref[0]
    d = ((q[:, 0:1] - p[0:1, :]) ** 2 + (q[:, 1:2] - p[1:2, :]) ** 2) \
        + (q[:, 2:3] - p[2:3, :]) ** 2
    iota = lax.broadcasted_iota(jnp.int32, d.shape, 1)
    cols = []
    for _ in range(KS):
        m = jnp.min(d, axis=1, keepdims=True)
        ij = jnp.min(jnp.where(d == m, iota, n), axis=1, keepdims=True)
        cols.append(ij)
        d = jnp.where(iota == ij, jnp.inf, d)
    o_ref[0] = jnp.concatenate(cols, axis=1)


def _knn(posT, posL):
    b, n, _ = posT.shape
    bq = min(n, 256)
    return pl.pallas_call(
        functools.partial(_knn_body, n=n),
        grid=(b, n // bq),
        in_specs=[pl.BlockSpec((1, bq, 3), lambda i, j: (i, j, 0)),
                  pl.BlockSpec((1, 3, n), lambda i, j: (i, 0, 0))],
        out_specs=pl.BlockSpec((1, bq, KS), lambda i, j: (i, j, 0)),
        out_shape=jax.ShapeDtypeStruct((b, n, KS), jnp.int32),
    )(posT, posL)


def _sc_gather(table, idx):
    r, d = table.shape
    (m,) = idx.shape
    info = plsc.get_sparse_core_info()
    nw = info.num_cores * info.num_subcores
    bpw = m // nw
    chunk = min(bpw, 128)
    while chunk * d * 4 > 196608:
        chunk //= 2
    n_iter = bpw // chunk
    mesh = plsc.VectorSubcoreMesh(core_axis_name="c", subcore_axis_name="s")

    @functools.partial(
        pl.kernel, mesh=mesh,
        out_type=jax.ShapeDtypeStruct((m, d), jnp.float32),
        scratch_types=[
            pltpu.VMEM((chunk,), jnp.int32),
            pltpu.VMEM((chunk, d), jnp.float32),
            pltpu.SemaphoreType.DMA,
        ],
    )
    def k(table_hbm, idx_hbm, out_hbm, idx_v, rows_v, sem):
        wid = lax.axis_index("s") * info.num_cores + lax.axis_index("c")

        def body(i, carry):
            base = wid * bpw + i * chunk
            pltpu.sync_copy(idx_hbm.at[pl.ds(base, chunk)], idx_v)
            pltpu.async_copy(table_hbm.at[idx_v], rows_v, sem).wait()
            pltpu.sync_copy(rows_v, out_hbm.at[pl.ds(base, chunk)])
            return carry

        lax.fori_loop(0, n_iter, body, 0)

    return k(table, idx)


def _pw_body(x_ref, w_ref, b_ref, o_ref, *, relu):
    y = jnp.dot(x_ref[0], w_ref[...], preferred_element_type=jnp.float32)
    y = y + b_ref[...]
    if relu:
        y = jnp.maximum(y, 0.0)
    o_ref[0] = y


def _pw(x, w, bias, relu):
    b, n, cin = x.shape
    cout = w.shape[1]
    bn_ = min(n, 512)
    ob = cout if cin * cout * 4 <= (6 << 20) else 128
    f = pl.pallas_call(
        functools.partial(_pw_body, relu=relu),
        grid=(b, n // bn_, cout // ob),
        in_specs=[pl.BlockSpec((1, bn_, cin), lambda i, j, o: (i, j, 0)),
                  pl.BlockSpec((cin, ob), lambda i, j, o: (0, o)),
                  pl.BlockSpec((1, ob), lambda i, j, o: (0, o))],
        out_specs=pl.BlockSpec((1, bn_, ob), lambda i, j, o: (i, j, o)),
        out_shape=jax.ShapeDtypeStruct((b, n, cout), jnp.float32),
    )
    return f(x, w, bias.reshape(1, cout))


def _pw_table_body(x_ref, w_ref, b_ref, p_ref, o_ref, *, pad):
    y = jnp.dot(x_ref[0], w_ref[...], preferred_element_type=jnp.float32)
    y = jnp.maximum(y + b_ref[...], 0.0)
    parts = [p_ref[0], y]
    if pad:
        parts.append(jnp.zeros((y.shape[0], pad), jnp.float32))
    o_ref[0] = jnp.concatenate(parts, axis=1)


def _pw_table(x, w, bias, posT):
    b, n, cin = x.shape
    cout = w.shape[1]
    dpad = _pad128(3 + cout)
    bn_ = min(n, 512)
    f = pl.pallas_call(
        functools.partial(_pw_table_body, pad=dpad - 3 - cout),
        grid=(b, n // bn_),
        in_specs=[pl.BlockSpec((1, bn_, cin), lambda i, j: (i, j, 0)),
                  pl.BlockSpec((cin, cout), lambda i, j: (0, 0)),
                  pl.BlockSpec((1, cout), lambda i, j: (0, 0)),
                  pl.BlockSpec((1, bn_, 3), lambda i, j: (i, j, 0))],
        out_specs=pl.BlockSpec((1, bn_, dpad), lambda i, j: (i, j, 0)),
        out_shape=jax.ShapeDtypeStruct((b, n, dpad), jnp.float32),
    )
    return f(x, w, bias.reshape(1, cout), posT)


def _pw_res_body(x_ref, w_ref, b_ref, r_ref, o_ref, *, pool, bn_):
    y = jnp.dot(x_ref[0], w_ref[...], preferred_element_type=jnp.float32)
    y = y + b_ref[...]
    r = r_ref[0]
    if pool:
        r = jnp.max(r.reshape(bn_, KS, r.shape[1]), axis=1)
    o_ref[0] = jnp.maximum(y + r, 0.0)


def _pw_res(x, w, bias, res, pool):
    b, n, cin = x.shape
    cout = w.shape[1]
    bn_ = min(n, 256 if pool else 512)
    rspec = (pl.BlockSpec((1, bn_ * KS, cout), lambda i, j: (i, j, 0)) if pool
             else pl.BlockSpec((1, bn_, cout), lambda i, j: (i, j, 0)))
    f = pl.pallas_call(
        functools.partial(_pw_res_body, pool=pool, bn_=bn_),
        grid=(b, n // bn_),
        in_specs=[pl.BlockSpec((1, bn_, cin), lambda i, j: (i, j, 0)),
                  pl.BlockSpec((cin, cout), lambda i, j: (0, 0)),
                  pl.BlockSpec((1, cout), lambda i, j: (0, 0)),
                  rspec],
        out_specs=pl.BlockSpec((1, bn_, cout), lambda i, j: (i, j, 0)),
        out_shape=jax.ShapeDtypeStruct((b, n, cout), jnp.float32),
    )
    return f(x, w, bias.reshape(1, cout), res)


def _fka_a_body(g_ref, sup_ref, ab_ref, fc1_ref, m1_ref, dw_ref, s_ref, q_ref,
                *, bs, dpad):
    nb = pl.program_id(1)
    g = g_ref[0].reshape(bs, KS, dpad)
    pts = g[:, :, 0:3] - sup_ref[0][:, None, :]
    d = jnp.sqrt(jnp.sum(pts * pts, axis=2))
    alpha = ab_ref[0, 0]
    beta = ab_ref[0, 1]
    dwr = 1.0 / (1.0 + jnp.exp(alpha * d - beta))
    dws = jnp.sum(dwr, axis=1, keepdims=True)
    dws = dws + (dws == 0).astype(jnp.float32) + 1e-6
    dwn = dwr / dws * float(KS)
    mat1 = jnp.dot(pts.reshape(bs * KS, 3), fc1_ref[...],
                   preferred_element_type=jnp.float32)
    m1_ref[0] = mat1
    dw_ref[0] = dwn

    @pl.when(nb == 0)
    def _():
        s_ref[...] = jnp.zeros_like(s_ref)
        q_ref[...] = jnp.zeros_like(q_ref)

    s_ref[...] += jnp.sum(mat1, axis=0).reshape(1, 1, 16)
    q_ref[...] += jnp.sum(mat1 * mat1, axis=0).reshape(1, 1, 16)


def _fka_b_body(m1_ref, dw_ref, s_ref, q_ref, fc2_ref, gb_ref, m2_ref,
                s2_ref, q2_ref, *, bs, cnt):
    nb = pl.program_id(1)
    m1 = m1_ref[0].reshape(bs, KS, 16)
    dwn = dw_ref[0]
    mean = (s_ref[0, 0] / cnt).reshape(1, 1, 16)
    var = (q_ref[0, 0] / cnt).reshape(1, 1, 16) - mean * mean
    g = gb_ref[0:1, :].reshape(1, 1, 16)
    b = gb_ref[1:2, :].reshape(1, 1, 16)
    m1n = jnp.maximum((m1 - mean) / jnp.sqrt(var + 1e-5) * g + b, 0.0)
    mp = jnp.max(m1n * dwn[:, :, None], axis=1, keepdims=True)
    cat = jnp.concatenate([m1n, jnp.broadcast_to(mp, (bs, KS, 16))], axis=2)
    mat2 = jnp.dot(cat.reshape(bs * KS, 32), fc2_ref[...],
                   preferred_element_type=jnp.float32)
    m2_ref[0] = mat2

    @pl.when(nb == 0)
    def _():
        s2_ref[...] = jnp.zeros_like(s2_ref)
        q2_ref[...] = jnp.zeros_like(q2_ref)

    s2_ref[...] += jnp.sum(mat2, axis=0).reshape(1, 1, 16)
    q2_ref[...] += jnp.sum(mat2 * mat2, axis=0).reshape(1, 1, 16)


def _fka_c_body(m2_ref, dw_ref, s2_ref, q2_ref, g_ref, fc3_ref, gb_ref, f_ref,
                *, bs, cc, dpad, cnt):
    m2 = m2_ref[0].reshape(bs, KS, 16)
    dwn = dw_ref[0]
    mean = (s2_ref[0, 0] / cnt).reshape(1, 1, 16)
    var = (q2_ref[0, 0] / cnt).reshape(1, 1, 16) - mean * mean
    g = gb_ref[0:1, :].reshape(1, 1, 16)
    b = gb_ref[1:2, :].reshape(1, 1, 16)
    m2n = jnp.maximum((m2 - mean) / jnp.sqrt(var + 1e-5) * g + b, 0.0)
    mp = jnp.max(m2n * dwn[:, :, None], axis=1, keepdims=True)
    cat = jnp.concatenate([m2n, jnp.broadcast_to(mp, (bs, KS, 16))], axis=2)
    mat3 = jnp.maximum(
        jnp.dot(cat.reshape(bs * KS, 32), fc3_ref[...],
                preferred_element_type=jnp.float32), 0.0)
    mat3 = mat3.reshape(bs, KS, 16) * dwn[:, :, None]
    xg = g_ref[0].reshape(bs, KS, dpad)[:, :, 3:3 + cc]
    acc = jnp.einsum("skj,skc->sjc", mat3, xg,
                     preferred_element_type=jnp.float32)
    f_ref[0] = acc.reshape(bs, 16 * cc)


def _fka(gtab, cc, supT, ab, fc1t, fc2t, gb1, fc3t, gb2, cvt, out_bias):
    b, sk, dpad = gtab.shape
    s_cnt = sk // KS
    bs = min(s_cnt, 256)
    grid = (b, s_cnt // bs)
    cnt = float(sk)

    m1, dwn, s1, q1 = pl.pallas_call(
        functools.partial(_fka_a_body, bs=bs, dpad=dpad),
        grid=grid,
        in_specs=[pl.BlockSpec((1, bs * KS, dpad), lambda i, j: (i, j, 0)),
                  pl.BlockSpec((1, bs, 3), lambda i, j: (i, j, 0)),
                  pl.BlockSpec((1, 2), lambda i, j: (0, 0)),
                  pl.BlockSpec((3, 16), lambda i, j: (0, 0))],
        out_specs=[pl.BlockSpec((1, bs * KS, 16), lambda i, j: (i, j, 0)),
                   pl.BlockSpec((1, bs, KS), lambda i, j: (i, j, 0)),
                   pl.BlockSpec((1, 1, 16), lambda i, j: (i, 0, 0)),
                   pl.BlockSpec((1, 1, 16), lambda i, j: (i, 0, 0))],
        out_shape=[jax.ShapeDtypeStruct((b, sk, 16), jnp.float32),
                   jax.ShapeDtypeStruct((b, s_cnt, KS), jnp.float32),
                   jax.ShapeDtypeStruct((b, 1, 16), jnp.float32),
                   jax.ShapeDtypeStruct((b, 1, 16), jnp.float32)],
    )(gtab, supT, ab, fc1t)

    m2, s2, q2 = pl.pallas_call(
        functools.partial(_fka_b_body, bs=bs, cnt=cnt),
        grid=grid,
        in_specs=[pl.BlockSpec((1, bs * KS, 16), lambda i, j: (i, j, 0)),
                  pl.BlockSpec((1, bs, KS), lambda i, j: (i, j, 0)),
                  pl.BlockSpec((1, 1, 16), lambda i, j: (i, 0, 0)),
                  pl.BlockSpec((1, 1, 16), lambda i, j: (i, 0, 0)),
                  pl.BlockSpec((32, 16), lambda i, j: (0, 0)),
                  pl.BlockSpec((2, 16), lambda i, j: (0, 0))],
        out_specs=[pl.BlockSpec((1, bs * KS, 16), lambda i, j: (i, j, 0)),
                   pl.BlockSpec((1, 1, 16), lambda i, j: (i, 0, 0)),
                   pl.BlockSpec((1, 1, 16), lambda i, j: (i, 0, 0))],
        out_shape=[jax.ShapeDtypeStruct((b, sk, 16), jnp.float32),
                   jax.ShapeDtypeStruct((b, 1, 16), jnp.float32),
                   jax.ShapeDtypeStruct((b, 1, 16), jnp.float32)],
    )(m1, dwn, s1, q1, fc2t, gb1)

    feats = pl.pallas_call(
        functools.partial(_fka_c_body, bs=bs, cc=cc, dpad=dpad, cnt=cnt),
        grid=grid,
        in_specs=[pl.BlockSpec((1, bs * KS, 16), lambda i, j: (i, j, 0)),
                  pl.BlockSpec((1, bs, KS), lambda i, j: (i, j, 0)),
                  pl.BlockSpec((1, 1, 16), lambda i, j: (i, 0, 0)),
                  pl.BlockSpec((1, 1, 16), lambda i, j: (i, 0, 0)),
                  pl.BlockSpec((1, bs * KS, dpad), lambda i, j: (i, j, 0)),
                  pl.BlockSpec((32, 16), lambda i, j: (0, 0)),
                  pl.BlockSpec((2, 16), lambda i, j: (0, 0))],
        out_specs=pl.BlockSpec((1, bs, cc * 16), lambda i, j: (i, j, 0)),
        out_shape=jax.ShapeDtypeStruct((b, s_cnt, cc * 16), jnp.float32),
    )(m2, dwn, s2, q2, gtab, fc3t, gb2)

    return _pw(feats, cvt, out_bias, relu=True)


def _final_body(x_ref, w_ref, b_ref, o_ref):
    xm = jnp.mean(x_ref[0], axis=0, keepdims=True)
    o_ref[0] = jnp.dot(xm, w_ref[...],
                       preferred_element_type=jnp.float32) + b_ref[...]


def _final(x, w, bias):
    b, n, cin = x.shape
    cout = w.shape[1]
    f = pl.pallas_call(
        _final_body,
        grid=(b,),
        in_specs=[pl.BlockSpec((1, n, cin), lambda i: (i, 0, 0)),
                  pl.BlockSpec((cin, cout), lambda i: (0, 0)),
                  pl.BlockSpec((1, cout), lambda i: (0, 0))],
        out_specs=pl.BlockSpec((1, 1, cout), lambda i: (i, 0, 0)),
        out_shape=jax.ShapeDtypeStruct((b, 1, cout), jnp.float32),
    )
    return f(x, w, bias.reshape(1, cout)).reshape(b, cout)


_BN_S = float(np.sqrt(1.0 + 1e-5))


def _fold(w, g, bb, cb=None):
    sc = g / _BN_S
    wt = (w * sc[:, None]).T
    bias = bb if cb is None else cb * sc + bb
    return wt, bias


def _fka_params(p, out_g, out_b):
    cout, cc, _ = p['cv_w'].shape
    sc = out_g / _BN_S
    cvt = (p['cv_w'] * sc[:, None, None]).transpose(0, 2, 1).reshape(
        cout, 16 * cc).T
    return dict(
        ab=jnp.stack([p['alpha'], p['beta']]).reshape(1, 2),
        fc1t=p['fc1_w'].T, fc2t=p['fc2_w'].T, fc3t=p['fc3_w'].T,
        gb1=jnp.stack([p['in1_g'], p['in1_b']]),
        gb2=jnp.stack([p['in2_g'], p['in2_b']]),
        cvt=cvt, out_bias=out_b, cc=cc)


def _run_fka(gtab, supT, fp):
    return _fka(gtab, fp['cc'], supT, fp['ab'], fp['fc1t'], fp['fc2t'],
                fp['gb1'], fp['fc3t'], fp['gb2'], fp['cvt'], fp['out_bias'])


def _gather_rows(table, idx):
    b, n, d = table.shape
    s = idx.shape[1]
    off = (jnp.arange(b, dtype=jnp.int32) * n)[:, None, None]
    flat = (idx + off).reshape(-1)
    return _sc_gather(table.reshape(b * n, d), flat).reshape(b, s * KS, d)


def _res_block(x, posT_in, idx, p, down):
    s_cnt = idx.shape[1]
    supT = posT_in[:, :s_cnt, :]
    w0t, b0 = _fold(p['cv0_w'], p['bn0_g'], p['bn0_b'], p['cv0_b'])
    table = _pw_table(x, w0t, b0, posT_in)
    gtab = _gather_rows(table, idx)
    fp = _fka_params(p['fka'], p['bn1_g'], p['bn1_b'])
    h2 = _run_fka(gtab, supT, fp)
    w2t, b2 = _fold(p['cv2_w'], p['bn2_g'], p['bn2_b'], p['cv2_b'])
    if down:
        wst, bs_ = _fold(p['sc_w'], p['bnsc_g'], p['bnsc_b'], p['sc_b'])
        xsc = _pw(x, wst, bs_, relu=False)
        xsc_g = _gather_rows(xsc, idx)
        return _pw_res(h2, w2t, b2, xsc_g, pool=True)
    return _pw_res(h2, w2t, b2, x, pool=False)


def kernel(x, pos, params):
    b = x.shape[0]
    n = pos.shape[2]
    posT = pos.transpose(0, 2, 1)
    npts = [n, n // 4, n // 16, n // 64, n // 256]

    ids = [_knn(posT[:, :nl, :], pos[:, :, :nl]) for nl in npts]

    xT = x.transpose(0, 2, 1)
    d0 = _pad128(6)
    table0 = jnp.concatenate(
        [posT, xT, jnp.zeros((b, n, d0 - 6), jnp.float32)], axis=2)
    g0 = _gather_rows(table0, ids[0])
    fp0 = _fka_params(params['cv0'], params['bn0_g'], params['bn0_b'])
    x0 = _run_fka(g0, posT, fp0)

    x0 = _res_block(x0, posT, ids[0], params['rb01'], down=False)
    x1 = _res_block(x0, posT, ids[0][:, :npts[1]], params['rb10'], down=True)
    x1 = _res_block(x1, posT[:, :npts[1]], ids[1], params['rb11'], down=False)
    x2 = _res_block(x1, posT[:, :npts[1]], ids[1][:, :npts[2]],
                    params['rb20'], down=True)
    x2 = _res_block(x2, posT[:, :npts[2]], ids[2], params['rb21'], down=False)
    x3 = _res_block(x2, posT[:, :npts[2]], ids[2][:, :npts[3]],
                    params['rb30'], down=True)
    x3 = _res_block(x3, posT[:, :npts[3]], ids[3], params['rb31'], down=False)
    x4 = _res_block(x3, posT[:, :npts[3]], ids[3][:, :npts[4]],
                    params['rb40'], down=True)
    x4 = _res_block(x4, posT[:, :npts[4]], ids[4], params['rb41'], down=False)

    return _final(x4, params['fcout_w'].T, params['fcout_b'])
